# baseline (speedup 1.0000x reference)
import functools

import jax
import jax.numpy as jnp
from jax import lax
from jax.experimental import pallas as pl
from jax.experimental.pallas import tpu as pltpu
from jax.experimental.pallas import tpu_sc as plsc

N_Z = 16
BATCH = 16384
NC = 2
NS = 16
NW = NC * NS
B_PER_W = BATCH // NW
IDX_CHUNK = 128
N_CHUNKS = B_PER_W // IDX_CHUNK
N_TILES = B_PER_W // N_Z


def _sc_body(x_hbm, idx_hbm, emb1_hbm, emb2_hbm, out_hbm,
             idx_v, x_v, a_v, b_v, o_v, sem1, sem2):
    wid = lax.axis_index("s") * NC + lax.axis_index("c")
    base = wid * B_PER_W

    pltpu.sync_copy(idx_hbm.at[pl.ds(base, B_PER_W)], idx_v)

    copies = []
    for g in range(N_CHUNKS):
        sl = pl.ds(g * IDX_CHUNK, IDX_CHUNK)
        copies.append(pltpu.async_copy(
            emb1_hbm.at[idx_v.at[sl]], a_v.at[sl], sem1))
        copies.append(pltpu.async_copy(
            emb2_hbm.at[idx_v.at[sl]], b_v.at[sl], sem2))
    pltpu.sync_copy(x_hbm.at[pl.ds(base, B_PER_W)], x_v)
    for c in copies:
        c.wait()

    lanes = lax.iota(jnp.int32, N_Z)

    def tile_body(t, _):
        r0 = t * N_Z
        res = b_v[pl.ds(r0, N_Z)]
        for r in range(N_Z):
            p = x_v[r0 + r] * a_v[r0 + r]
            res = jnp.where(lanes == r, res + jnp.sum(p), res)
        o_v[pl.ds(r0, N_Z)] = res
        return 0

    lax.fori_loop(0, N_TILES, tile_body, 0)
    pltpu.sync_copy(o_v, out_hbm.at[pl.ds(base, B_PER_W)])


@jax.jit
def _rand_effect(x, idx, emb1, emb2):
    mesh = plsc.VectorSubcoreMesh(core_axis_name="c", subcore_axis_name="s")
    k = functools.partial(
        pl.kernel,
        out_type=jax.ShapeDtypeStruct((BATCH,), jnp.float32),
        mesh=mesh,
        compiler_params=pltpu.CompilerParams(
            needs_layout_passes=False, use_tc_tiling_on_sc=False),
        scratch_types=[
            pltpu.VMEM((B_PER_W,), jnp.int32),
            pltpu.VMEM((B_PER_W, N_Z), jnp.float32),
            pltpu.VMEM((B_PER_W, N_Z), jnp.float32),
            pltpu.VMEM((B_PER_W,), jnp.float32),
            pltpu.VMEM((B_PER_W,), jnp.float32),
            pltpu.SemaphoreType.DMA,
            pltpu.SemaphoreType.DMA,
        ],
    )(_sc_body)
    return k(x, idx, emb1, emb2)


def kernel(x, idx, emb1, emb2):
    out = _rand_effect(x, idx.astype(jnp.int32), emb1, emb2.reshape(-1))
    return out.reshape(BATCH, 1)
